# baseline (speedup 1.0000x reference)
import functools

import jax
import jax.numpy as jnp
from jax import lax
from jax.experimental import pallas as pl
from jax.experimental.pallas import tpu as pltpu
from jax.experimental.pallas import tpu_sc as plsc

N = 10000
D = 128
DH = 64
H = 8
HL = 4
C = 16
NEG = 0.2
NC, NS, L = 2, 16, 16
CHUNK = 64
NPAD = 10016
NXL = 10016
ROWS_PER_TILE = NPAD // NS


def _proj_body(x_ref, w_ref, out_ref):
    x = x_ref[...]
    w = w_ref[0]
    out_ref[0] = jnp.dot(x, w, preferred_element_type=jnp.float32)


def _project(xpad, wcat):
    blk = 2504
    grid_i = NXL // blk
    out = pl.pallas_call(
        _proj_body,
        grid=(4, grid_i),
        in_specs=[
            pl.BlockSpec((blk, D), lambda q, i: (i, 0)),
            pl.BlockSpec((1, D, DH), lambda q, i: (q, 0, 0)),
        ],
        out_specs=pl.BlockSpec((1, blk, DH), lambda q, i: (q, i, 0)),
        out_shape=jax.ShapeDtypeStruct((4, NXL, DH), jnp.float32),
    )(xpad, wcat)
    return out.reshape(4 * NXL, DH)


def _make_edge_kernel(cpt):
    mesh = plsc.VectorSubcoreMesh(
        core_axis_name="c", subcore_axis_name="s", num_cores=NC, num_subcores=NS
    )

    @functools.partial(
        pl.kernel,
        mesh=mesh,
        compiler_params=pltpu.CompilerParams(use_tc_tiling_on_sc=False),
        out_type=jax.ShapeDtypeStruct((NC, NPAD, DH), jnp.float32),
        scratch_types=[
            pltpu.VMEM((cpt * CHUNK,), jnp.int32),
            pltpu.VMEM((cpt * CHUNK,), jnp.int32),
            pltpu.VMEM((CHUNK,), jnp.int32),
            pltpu.VMEM((CHUNK,), jnp.int32),
            pltpu.VMEM((CHUNK, DH), jnp.float32),
            pltpu.VMEM((CHUNK, DH), jnp.float32),
            pltpu.VMEM((CHUNK, DH), jnp.float32),
            pltpu.VMEM((CHUNK, DH), jnp.float32),
            pltpu.VMEM((CHUNK, DH), jnp.float32),
            pltpu.VMEM((CHUNK, DH), jnp.float32),
            pltpu.VMEM((CHUNK, L), jnp.float32),
            pltpu.VMEM((CHUNK, L), jnp.float32),
            pltpu.VMEM((CHUNK,), jnp.int32),
            pltpu.VMEM((CHUNK,), jnp.int32),
            pltpu.VMEM((HL, L), jnp.float32),
            pltpu.VMEM((HL + 1, L), jnp.float32),
            pltpu.VMEM((HL, L), jnp.float32),
            pltpu.VMEM((2, L), jnp.int32),
            pltpu.VMEM_SHARED((NPAD, DH), jnp.float32),
            pltpu.VMEM_SHARED((NPAD, L), jnp.float32),
            pltpu.SemaphoreType.DMA,
            pltpu.SemaphoreType.DMA,
            pltpu.SemaphoreType.DMA,
            pltpu.SemaphoreType.DMA,
            pltpu.SemaphoreType.DMA,
            pltpu.SemaphoreType.DMA,
            pltpu.SemaphoreType.DMA,
            pltpu.SemaphoreType.DMA,
        ],
    )
    def edge_kernel(tab_hbm, edges_hbm, att_hbm, hot_hbm, bias_hbm,
                    offs_hbm, out_hbm,
                    big_s, big_d, db0, db1, xl0, xl1, xr0, xr1, msg0, msg1,
                    exb0, exb1, da0, da1, att_buf, hot_buf, bias_buf, off_buf,
                    numer_sh, denom_sh,
                    gs0, gs1, gd0, gd1, sm0, sm1, se0, se1):
        cid = lax.axis_index("c")
        sid = lax.axis_index("s")
        row0 = sid * ROWS_PER_TILE

        xl_b = (xl0, xl1)
        xr_b = (xr0, xr1)
        msg_b = (msg0, msg1)
        ex_b = (exb0, exb1)
        da_b = (da0, da1)
        db_b = (db0, db1)
        gs_b = (gs0, gs1)
        gd_b = (gd0, gd1)
        sm_b = (sm0, sm1)
        se_b = (se0, se1)

        zeros16 = jnp.zeros((L,), jnp.float32)
        negv = jnp.full((L,), NEG, jnp.float32)
        lane = lax.iota(jnp.int32, L)
        p8, p4, p2, p1 = [lane ^ (1 << p) for p in (3, 2, 1, 0)]
        splat_pat = [jnp.full((L,), 4 * k, jnp.int32) for k in range(HL)]
        compact_pat = (lane & 3) * 4

        def zbody(r, carry):
            for g in range(DH // L):
                msg0[r, pl.ds(g * L, L)] = zeros16
            exb0[r, :] = zeros16
            return carry

        lax.fori_loop(0, CHUNK, zbody, 0)

        npc = ROWS_PER_TILE // CHUNK
        pieces = [CHUNK] * npc
        if ROWS_PER_TILE % CHUNK:
            pieces.append(ROWS_PER_TILE % CHUNK)
        zdone = 0
        for sz in pieces:
            pltpu.sync_copy(msg0.at[pl.ds(0, sz)],
                            numer_sh.at[pl.ds(row0 + zdone, sz)])
            pltpu.sync_copy(exb0.at[pl.ds(0, sz)],
                            denom_sh.at[pl.ds(row0 + zdone, sz)])
            zdone += sz
        plsc.subcore_barrier()

        pltpu.sync_copy(att_hbm.at[cid], att_buf)
        pltpu.sync_copy(hot_hbm, hot_buf)
        pltpu.sync_copy(bias_hbm.at[cid], bias_buf)
        pltpu.sync_copy(offs_hbm.at[cid], off_buf)
        att_vecs = [att_buf[k, :] for k in range(HL)]
        grp_vecs = [hot_buf[k, :] for k in range(HL)]
        lo4_vec = hot_buf[HL, :]
        off_s = off_buf[0, :]
        off_d = off_buf[1, :]

        ept = cpt * CHUNK
        pltpu.sync_copy(edges_hbm.at[pl.ds(sid * ept, ept)], big_s)
        maskv = jnp.full((L,), 0x3FFF, jnp.int32)

        def adj_body(r, carry):
            sl = pl.ds(r * L, L)
            v = big_s[sl]
            big_d[sl] = lax.shift_right_logical(v, 14)
            big_s[sl] = (v & maskv) + off_s
            return carry

        lax.fori_loop(0, ept // L, adj_body, 0)

        def fill_da(b, j):
            for g in range(CHUNK // L):
                sl = pl.ds(g * L, L)
                da_b[b][sl] = big_d[pl.ds(j * CHUNK + g * L, L)] + off_d

        def fill_db(b, j):
            for g in range(CHUNK // L):
                sl = pl.ds(g * L, L)
                db_b[b][sl] = big_d[pl.ds(j * CHUNK + g * L, L)]

        def issue_gathers(b, j):
            fill_da(b, j)
            pltpu.async_copy(tab_hbm.at[big_s.at[pl.ds(j * CHUNK, CHUNK)]],
                             xl_b[b], gs_b[b])
            pltpu.async_copy(tab_hbm.at[da_b[b]], xr_b[b], gd_b[b])

        def wait_gathers(b):
            pltpu.make_async_copy(tab_hbm.at[da_b[b]], xl_b[b], gs_b[b]).wait()
            pltpu.make_async_copy(tab_hbm.at[da_b[b]], xr_b[b], gd_b[b]).wait()

        def wait_scatters(b):
            pltpu.make_async_copy(msg_b[b], numer_sh.at[db_b[b]], sm_b[b]).wait()
            pltpu.make_async_copy(ex_b[b], denom_sh.at[db_b[b]], se_b[b]).wait()

        def compute_chunk(b):
            xlb, xrb, msgb, exb = xl_b[b], xr_b[b], msg_b[b], ex_b[b]

            def edge_body(e, ecarry):
                als, rs = [], []
                for k in range(HL):
                    a = xlb[e, pl.ds(k * L, L)]
                    bb = xrb[e, pl.ds(k * L, L)]
                    s = a + bb
                    s = jnp.maximum(s, zeros16) + negv * jnp.minimum(s, zeros16)
                    t = s * att_vecs[k]
                    t = t + t.at[p8].get(mode="promise_in_bounds")
                    t = t + t.at[p4].get(mode="promise_in_bounds")
                    als.append(a)
                    rs.append(t)
                z = (rs[0] * grp_vecs[0] + rs[1] * grp_vecs[1]
                     + rs[2] * grp_vecs[2] + rs[3] * grp_vecs[3])
                z = z + z.at[p2].get(mode="promise_in_bounds")
                z = z + z.at[p1].get(mode="promise_in_bounds")
                e_all = jnp.exp(z)
                for k in range(HL):
                    sk = e_all.at[splat_pat[k]].get(mode="promise_in_bounds")
                    msgb[e, pl.ds(k * L, L)] = sk * als[k]
                exb[e, :] = (e_all.at[compact_pat].get(mode="promise_in_bounds")
                             * lo4_vec)
                return ecarry

            lax.fori_loop(0, CHUNK, edge_body, 0)

        issue_gathers(0, 0)
        issue_gathers(1, jnp.int32(1))

        def pipe_body(i, carry):
            for b in range(2):
                j = 2 * i + b
                wait_gathers(b)

                @pl.when(i >= 1)
                def _():
                    wait_scatters(b)

                compute_chunk(b)
                fill_db(b, j)
                pltpu.async_copy(msg_b[b], numer_sh.at[db_b[b]], sm_b[b],
                                 add=True)
                pltpu.async_copy(ex_b[b], denom_sh.at[db_b[b]], se_b[b],
                                 add=True)
                jn = jnp.minimum(j + 2, cpt - 1)
                issue_gathers(b, jn)
            return carry

        lax.fori_loop(0, cpt // 2, pipe_body, 0)
        for b in range(2):
            wait_gathers(b)
            wait_scatters(b)

        plsc.subcore_barrier()

        bias_vecs = [bias_buf[k, :] for k in range(HL)]
        epsv = jnp.full((L,), 1e-16, jnp.float32)
        idxk = [jnp.full((L,), k, jnp.int32) for k in range(HL)]
        done = 0
        for sz in pieces:
            r0 = row0 + done
            pltpu.sync_copy(numer_sh.at[pl.ds(r0, sz)], xl0.at[pl.ds(0, sz)])
            pltpu.sync_copy(denom_sh.at[pl.ds(r0, sz)], exb0.at[pl.ds(0, sz)])

            def div_body(r, carry):
                den_row = exb0[r, :]
                for k in range(HL):
                    dk = den_row.at[idxk[k]].get(mode="promise_in_bounds")
                    sl = pl.ds(k * L, L)
                    msg0[r, sl] = xl0[r, sl] / (dk + epsv) + bias_vecs[k]
                return carry

            lax.fori_loop(0, sz, div_body, 0)
            pltpu.sync_copy(msg0.at[pl.ds(0, sz)],
                            out_hbm.at[cid, pl.ds(r0, sz)])
            done += sz

    return edge_kernel


def kernel(x, edge_index, W_l, W_r, att, bias):
    E2 = edge_index.shape[1]
    etot = E2 + N
    loop = jnp.arange(N, dtype=edge_index.dtype)
    src = jnp.concatenate([edge_index[0], loop])
    dst = jnp.concatenate([edge_index[1], loop])

    edges_per_tile = -(-etot // (NS * 2 * CHUNK)) * 2 * CHUNK
    cpt = edges_per_tile // CHUNK
    epad = edges_per_tile * NS
    src = jnp.pad(src, (0, epad - etot), constant_values=N)
    dst = jnp.pad(dst, (0, epad - etot), constant_values=N)
    edges_packed = src | (dst << 14)

    xpad = jnp.pad(x, ((0, NXL - N), (0, 0)))
    wcat = jnp.stack([W_l[:, :DH], W_l[:, DH:], W_r[:, :DH], W_r[:, DH:]])
    tab = _project(xpad, wcat)

    att_g = att.reshape(NC, HL, C)
    bias_g = bias.astype(jnp.float32).reshape(NC, HL, C)
    lane_ids = jnp.arange(L)
    grp = (lane_ids[None, :] // 4 == jnp.arange(HL)[:, None]).astype(jnp.float32)
    lo4 = (lane_ids < HL).astype(jnp.float32)[None, :]
    hot = jnp.concatenate([grp, lo4], axis=0)
    offs = jnp.stack([
        jnp.full((2, L), 0 * NXL, jnp.int32) + jnp.array([[0], [2 * NXL]], jnp.int32),
        jnp.full((2, L), 1 * NXL, jnp.int32) + jnp.array([[0], [2 * NXL]], jnp.int32),
    ])

    edge_kernel = _make_edge_kernel(cpt)
    out_halves = edge_kernel(tab, edges_packed, att_g, hot, bias_g, offs)

    return out_halves.transpose(1, 0, 2).reshape(NPAD, D)[:N]


# speedup vs baseline: 1.0165x; 1.0165x over previous
import functools

import jax
import jax.numpy as jnp
from jax import lax
from jax.experimental import pallas as pl
from jax.experimental.pallas import tpu as pltpu
from jax.experimental.pallas import tpu_sc as plsc

N = 10000
D = 128
DH = 64
H = 8
HL = 4
C = 16
NEG = 0.2
NC, NS, L = 2, 16, 16
CHUNK = 64
NPAD = 10016
NXL = 10016
ROWS_PER_TILE = NPAD // NS


def _proj_body(x_ref, w_ref, out_ref):
    x = x_ref[...]
    w = w_ref[0]
    out_ref[0] = jnp.dot(x, w, preferred_element_type=jnp.float32)


def _project(xpad, wcat):
    blk = 2504
    grid_i = NXL // blk
    out = pl.pallas_call(
        _proj_body,
        grid=(4, grid_i),
        in_specs=[
            pl.BlockSpec((blk, D), lambda q, i: (i, 0)),
            pl.BlockSpec((1, D, DH), lambda q, i: (q, 0, 0)),
        ],
        out_specs=pl.BlockSpec((1, blk, DH), lambda q, i: (q, i, 0)),
        out_shape=jax.ShapeDtypeStruct((4, NXL, DH), jnp.float32),
    )(xpad, wcat)
    return out.reshape(4 * NXL, DH)


def _make_edge_kernel(cpt):
    mesh = plsc.VectorSubcoreMesh(
        core_axis_name="c", subcore_axis_name="s", num_cores=NC, num_subcores=NS
    )

    @functools.partial(
        pl.kernel,
        mesh=mesh,
        compiler_params=pltpu.CompilerParams(use_tc_tiling_on_sc=False),
        out_type=jax.ShapeDtypeStruct((NC, NPAD, DH), jnp.float32),
        scratch_types=[
            pltpu.VMEM((cpt * CHUNK,), jnp.int32),
            pltpu.VMEM((cpt * CHUNK,), jnp.int32),
            pltpu.VMEM((CHUNK,), jnp.int32),
            pltpu.VMEM((CHUNK,), jnp.int32),
            pltpu.VMEM((CHUNK, DH), jnp.float32),
            pltpu.VMEM((CHUNK, DH), jnp.float32),
            pltpu.VMEM((CHUNK, DH), jnp.float32),
            pltpu.VMEM((CHUNK, DH), jnp.float32),
            pltpu.VMEM((CHUNK, DH), jnp.float32),
            pltpu.VMEM((CHUNK, DH), jnp.float32),
            pltpu.VMEM((CHUNK, L), jnp.float32),
            pltpu.VMEM((CHUNK, L), jnp.float32),
            pltpu.VMEM((CHUNK,), jnp.int32),
            pltpu.VMEM((CHUNK,), jnp.int32),
            pltpu.VMEM((HL, L), jnp.float32),
            pltpu.VMEM((HL + 1, L), jnp.float32),
            pltpu.VMEM((HL, L), jnp.float32),
            pltpu.VMEM((2, L), jnp.int32),
            pltpu.VMEM_SHARED((NPAD, DH), jnp.float32),
            pltpu.VMEM_SHARED((NPAD, L), jnp.float32),
            pltpu.SemaphoreType.DMA,
            pltpu.SemaphoreType.DMA,
            pltpu.SemaphoreType.DMA,
            pltpu.SemaphoreType.DMA,
            pltpu.SemaphoreType.DMA,
            pltpu.SemaphoreType.DMA,
            pltpu.SemaphoreType.DMA,
            pltpu.SemaphoreType.DMA,
        ],
    )
    def edge_kernel(tab_hbm, edges_hbm, att_hbm, hot_hbm, bias_hbm,
                    offs_hbm, out_hbm,
                    big_s, big_d, db0, db1, xl0, xl1, xr0, xr1, msg0, msg1,
                    exb0, exb1, da0, da1, att_buf, hot_buf, bias_buf, off_buf,
                    numer_sh, denom_sh,
                    gs0, gs1, gd0, gd1, sm0, sm1, se0, se1):
        cid = lax.axis_index("c")
        sid = lax.axis_index("s")
        row0 = sid * ROWS_PER_TILE

        xl_b = (xl0, xl1)
        xr_b = (xr0, xr1)
        msg_b = (msg0, msg1)
        ex_b = (exb0, exb1)
        da_b = (da0, da1)
        db_b = (db0, db1)
        gs_b = (gs0, gs1)
        gd_b = (gd0, gd1)
        sm_b = (sm0, sm1)
        se_b = (se0, se1)

        zeros16 = jnp.zeros((L,), jnp.float32)
        negv = jnp.full((L,), NEG, jnp.float32)
        lane = lax.iota(jnp.int32, L)
        p8, p4, p2, p1 = [lane ^ (1 << p) for p in (3, 2, 1, 0)]
        splat_pat = [jnp.full((L,), 4 * k, jnp.int32) for k in range(HL)]
        compact_pat = (lane & 3) * 4

        def zbody(r, carry):
            for g in range(DH // L):
                msg0[r, pl.ds(g * L, L)] = zeros16
            exb0[r, :] = zeros16
            return carry

        lax.fori_loop(0, CHUNK, zbody, 0)

        npc = ROWS_PER_TILE // CHUNK
        pieces = [CHUNK] * npc
        if ROWS_PER_TILE % CHUNK:
            pieces.append(ROWS_PER_TILE % CHUNK)
        zdone = 0
        for sz in pieces:
            pltpu.sync_copy(msg0.at[pl.ds(0, sz)],
                            numer_sh.at[pl.ds(row0 + zdone, sz)])
            pltpu.sync_copy(exb0.at[pl.ds(0, sz)],
                            denom_sh.at[pl.ds(row0 + zdone, sz)])
            zdone += sz
        plsc.subcore_barrier()

        pltpu.sync_copy(att_hbm.at[cid], att_buf)
        pltpu.sync_copy(hot_hbm, hot_buf)
        pltpu.sync_copy(bias_hbm.at[cid], bias_buf)
        pltpu.sync_copy(offs_hbm.at[cid], off_buf)
        att_vecs = [att_buf[k, :] for k in range(HL)]
        grp_vecs = [hot_buf[k, :] for k in range(HL)]
        lo4_vec = hot_buf[HL, :]
        off_s = off_buf[0, :]
        off_d = off_buf[1, :]

        ept = cpt * CHUNK
        pltpu.sync_copy(edges_hbm.at[pl.ds(sid * ept, ept)], big_s)
        maskv = jnp.full((L,), 0x3FFF, jnp.int32)

        def adj_body(r, carry):
            sl = pl.ds(r * L, L)
            v = big_s[sl]
            big_d[sl] = lax.shift_right_logical(v, 14)
            big_s[sl] = (v & maskv) + off_s
            return carry

        lax.fori_loop(0, ept // L, adj_body, 0)

        def fill_da(b, j):
            for g in range(CHUNK // L):
                sl = pl.ds(g * L, L)
                da_b[b][sl] = big_d[pl.ds(j * CHUNK + g * L, L)] + off_d

        def fill_db(b, j):
            for g in range(CHUNK // L):
                sl = pl.ds(g * L, L)
                db_b[b][sl] = big_d[pl.ds(j * CHUNK + g * L, L)]

        def issue_gathers(b, j):
            fill_da(b, j)
            pltpu.async_copy(tab_hbm.at[big_s.at[pl.ds(j * CHUNK, CHUNK)]],
                             xl_b[b], gs_b[b])
            pltpu.async_copy(tab_hbm.at[da_b[b]], xr_b[b], gd_b[b])

        def wait_gathers(b):
            pltpu.make_async_copy(tab_hbm.at[da_b[b]], xl_b[b], gs_b[b]).wait()
            pltpu.make_async_copy(tab_hbm.at[da_b[b]], xr_b[b], gd_b[b]).wait()

        def wait_scatters(b):
            pltpu.make_async_copy(msg_b[b], numer_sh.at[db_b[b]], sm_b[b]).wait()
            pltpu.make_async_copy(ex_b[b], denom_sh.at[db_b[b]], se_b[b]).wait()

        def compute_chunk(b):
            xlb, xrb, msgb, exb = xl_b[b], xr_b[b], msg_b[b], ex_b[b]

            def edge_one(e):
                als, rs = [], []
                for k in range(HL):
                    a = xlb[e, pl.ds(k * L, L)]
                    bb = xrb[e, pl.ds(k * L, L)]
                    s = a + bb
                    s = jnp.maximum(s, zeros16) + negv * jnp.minimum(s, zeros16)
                    t = s * att_vecs[k]
                    t = t + t.at[p8].get(mode="promise_in_bounds")
                    t = t + t.at[p4].get(mode="promise_in_bounds")
                    als.append(a)
                    rs.append(t)
                z = (rs[0] * grp_vecs[0] + rs[1] * grp_vecs[1]
                     + rs[2] * grp_vecs[2] + rs[3] * grp_vecs[3])
                z = z + z.at[p2].get(mode="promise_in_bounds")
                z = z + z.at[p1].get(mode="promise_in_bounds")
                e_all = jnp.exp(z)
                for k in range(HL):
                    sk = e_all.at[splat_pat[k]].get(mode="promise_in_bounds")
                    msgb[e, pl.ds(k * L, L)] = sk * als[k]
                exb[e, :] = (e_all.at[compact_pat].get(mode="promise_in_bounds")
                             * lo4_vec)

            def edge_body(i, ecarry):
                edge_one(2 * i)
                edge_one(2 * i + 1)
                return ecarry

            lax.fori_loop(0, CHUNK // 2, edge_body, 0)

        issue_gathers(0, 0)
        issue_gathers(1, jnp.int32(1))

        def pipe_body(i, carry):
            for b in range(2):
                j = 2 * i + b
                wait_gathers(b)

                @pl.when(i >= 1)
                def _():
                    wait_scatters(b)

                compute_chunk(b)
                fill_db(b, j)
                pltpu.async_copy(msg_b[b], numer_sh.at[db_b[b]], sm_b[b],
                                 add=True)
                pltpu.async_copy(ex_b[b], denom_sh.at[db_b[b]], se_b[b],
                                 add=True)
                jn = jnp.minimum(j + 2, cpt - 1)
                issue_gathers(b, jn)
            return carry

        lax.fori_loop(0, cpt // 2, pipe_body, 0)
        for b in range(2):
            wait_gathers(b)
            wait_scatters(b)

        plsc.subcore_barrier()

        bias_vecs = [bias_buf[k, :] for k in range(HL)]
        epsv = jnp.full((L,), 1e-16, jnp.float32)
        idxk = [jnp.full((L,), k, jnp.int32) for k in range(HL)]
        done = 0
        for sz in pieces:
            r0 = row0 + done
            pltpu.sync_copy(numer_sh.at[pl.ds(r0, sz)], xl0.at[pl.ds(0, sz)])
            pltpu.sync_copy(denom_sh.at[pl.ds(r0, sz)], exb0.at[pl.ds(0, sz)])

            def div_body(r, carry):
                den_row = exb0[r, :]
                for k in range(HL):
                    dk = den_row.at[idxk[k]].get(mode="promise_in_bounds")
                    sl = pl.ds(k * L, L)
                    msg0[r, sl] = xl0[r, sl] / (dk + epsv) + bias_vecs[k]
                return carry

            lax.fori_loop(0, sz, div_body, 0)
            pltpu.sync_copy(msg0.at[pl.ds(0, sz)],
                            out_hbm.at[cid, pl.ds(r0, sz)])
            done += sz

    return edge_kernel


def kernel(x, edge_index, W_l, W_r, att, bias):
    E2 = edge_index.shape[1]
    etot = E2 + N
    loop = jnp.arange(N, dtype=edge_index.dtype)
    src = jnp.concatenate([edge_index[0], loop])
    dst = jnp.concatenate([edge_index[1], loop])

    edges_per_tile = -(-etot // (NS * 2 * CHUNK)) * 2 * CHUNK
    cpt = edges_per_tile // CHUNK
    epad = edges_per_tile * NS
    src = jnp.pad(src, (0, epad - etot), constant_values=N)
    dst = jnp.pad(dst, (0, epad - etot), constant_values=N)
    edges_packed = src | (dst << 14)

    xpad = jnp.pad(x, ((0, NXL - N), (0, 0)))
    wcat = jnp.stack([W_l[:, :DH], W_l[:, DH:], W_r[:, :DH], W_r[:, DH:]])
    tab = _project(xpad, wcat)

    att_g = att.reshape(NC, HL, C)
    bias_g = bias.astype(jnp.float32).reshape(NC, HL, C)
    lane_ids = jnp.arange(L)
    grp = (lane_ids[None, :] // 4 == jnp.arange(HL)[:, None]).astype(jnp.float32)
    lo4 = (lane_ids < HL).astype(jnp.float32)[None, :]
    hot = jnp.concatenate([grp, lo4], axis=0)
    offs = jnp.stack([
        jnp.full((2, L), 0 * NXL, jnp.int32) + jnp.array([[0], [2 * NXL]], jnp.int32),
        jnp.full((2, L), 1 * NXL, jnp.int32) + jnp.array([[0], [2 * NXL]], jnp.int32),
    ])

    edge_kernel = _make_edge_kernel(cpt)
    out_halves = edge_kernel(tab, edges_packed, att_g, hot, bias_g, offs)

    return out_halves.transpose(1, 0, 2).reshape(NPAD, D)[:N]


# speedup vs baseline: 1.0632x; 1.0460x over previous
import functools

import jax
import jax.numpy as jnp
from jax import lax
from jax.experimental import pallas as pl
from jax.experimental.pallas import tpu as pltpu
from jax.experimental.pallas import tpu_sc as plsc

N = 10000
D = 128
DH = 64
H = 8
HL = 4
C = 16
NEG = 0.2
NC, NS, L = 2, 16, 16
CHUNK = 64
NPAD = 10016
NXL = 10016
ROWS_PER_TILE = NPAD // NS


def _proj_body(x_ref, w_ref, out_ref):
    x = x_ref[...]
    w = w_ref[0]
    out_ref[0] = jnp.dot(x, w, preferred_element_type=jnp.float32)


def _project(xpad, wcat):
    blk = 2504
    grid_i = NXL // blk
    out = pl.pallas_call(
        _proj_body,
        grid=(4, grid_i),
        in_specs=[
            pl.BlockSpec((blk, D), lambda q, i: (i, 0)),
            pl.BlockSpec((1, D, DH), lambda q, i: (q, 0, 0)),
        ],
        out_specs=pl.BlockSpec((1, blk, DH), lambda q, i: (q, i, 0)),
        out_shape=jax.ShapeDtypeStruct((4, NXL, DH), jnp.float32),
    )(xpad, wcat)
    return out.reshape(4 * NXL, DH)


def _make_edge_kernel(cpt):
    mesh = plsc.VectorSubcoreMesh(
        core_axis_name="c", subcore_axis_name="s", num_cores=NC, num_subcores=NS
    )

    @functools.partial(
        pl.kernel,
        mesh=mesh,
        compiler_params=pltpu.CompilerParams(use_tc_tiling_on_sc=False),
        out_type=jax.ShapeDtypeStruct((NC, NPAD, DH), jnp.float32),
        scratch_types=[
            pltpu.VMEM((cpt * CHUNK,), jnp.int32),
            pltpu.VMEM((cpt * CHUNK,), jnp.int32),
            pltpu.VMEM((CHUNK,), jnp.int32),
            pltpu.VMEM((CHUNK,), jnp.int32),
            pltpu.VMEM((CHUNK, DH), jnp.float32),
            pltpu.VMEM((CHUNK, DH), jnp.float32),
            pltpu.VMEM((CHUNK, DH), jnp.float32),
            pltpu.VMEM((CHUNK, DH), jnp.float32),
            pltpu.VMEM((CHUNK, DH), jnp.float32),
            pltpu.VMEM((CHUNK, DH), jnp.float32),
            pltpu.VMEM((CHUNK, L), jnp.float32),
            pltpu.VMEM((CHUNK, L), jnp.float32),
            pltpu.VMEM((CHUNK,), jnp.int32),
            pltpu.VMEM((CHUNK,), jnp.int32),
            pltpu.VMEM((HL, L), jnp.float32),
            pltpu.VMEM((HL + 1, L), jnp.float32),
            pltpu.VMEM((HL, L), jnp.float32),
            pltpu.VMEM((2, L), jnp.int32),
            pltpu.VMEM_SHARED((NPAD, DH), jnp.float32),
            pltpu.VMEM_SHARED((NPAD, L), jnp.float32),
            pltpu.SemaphoreType.DMA,
            pltpu.SemaphoreType.DMA,
            pltpu.SemaphoreType.DMA,
            pltpu.SemaphoreType.DMA,
            pltpu.SemaphoreType.DMA,
            pltpu.SemaphoreType.DMA,
            pltpu.SemaphoreType.DMA,
            pltpu.SemaphoreType.DMA,
        ],
    )
    def edge_kernel(tab_hbm, edges_hbm, att_hbm, hot_hbm, bias_hbm,
                    offs_hbm, out_hbm,
                    big_s, big_d, db0, db1, xl0, xl1, xr0, xr1, msg0, msg1,
                    exb0, exb1, da0, da1, att_buf, hot_buf, bias_buf, off_buf,
                    numer_sh, denom_sh,
                    gs0, gs1, gd0, gd1, sm0, sm1, se0, se1):
        cid = lax.axis_index("c")
        sid = lax.axis_index("s")
        row0 = sid * ROWS_PER_TILE

        xl_b = (xl0, xl1)
        xr_b = (xr0, xr1)
        msg_b = (msg0, msg1)
        ex_b = (exb0, exb1)
        da_b = (da0, da1)
        db_b = (db0, db1)
        gs_b = (gs0, gs1)
        gd_b = (gd0, gd1)
        sm_b = (sm0, sm1)
        se_b = (se0, se1)

        zeros16 = jnp.zeros((L,), jnp.float32)
        negv = jnp.full((L,), NEG, jnp.float32)
        lane = lax.iota(jnp.int32, L)
        perms = [lane ^ (1 << p) for p in range(4)]
        splat_pat = [jnp.full((L,), k, jnp.int32) for k in range(HL)]

        def zbody(r, carry):
            for g in range(DH // L):
                msg0[r, pl.ds(g * L, L)] = zeros16
            exb0[r, :] = zeros16
            return carry

        lax.fori_loop(0, CHUNK, zbody, 0)

        npc = ROWS_PER_TILE // CHUNK
        pieces = [CHUNK] * npc
        if ROWS_PER_TILE % CHUNK:
            pieces.append(ROWS_PER_TILE % CHUNK)
        zdone = 0
        for sz in pieces:
            pltpu.sync_copy(msg0.at[pl.ds(0, sz)],
                            numer_sh.at[pl.ds(row0 + zdone, sz)])
            pltpu.sync_copy(exb0.at[pl.ds(0, sz)],
                            denom_sh.at[pl.ds(row0 + zdone, sz)])
            zdone += sz
        plsc.subcore_barrier()

        pltpu.sync_copy(att_hbm.at[cid], att_buf)
        pltpu.sync_copy(hot_hbm, hot_buf)
        pltpu.sync_copy(bias_hbm.at[cid], bias_buf)
        pltpu.sync_copy(offs_hbm.at[cid], off_buf)
        att_vecs = [att_buf[k, :] for k in range(HL)]
        hot_vecs = [hot_buf[k, :] for k in range(HL)]
        lo4_vec = hot_buf[HL, :]
        off_s = off_buf[0, :]
        off_d = off_buf[1, :]

        ept = cpt * CHUNK
        pltpu.sync_copy(edges_hbm.at[pl.ds(sid * ept, ept)], big_s)
        maskv = jnp.full((L,), 0x3FFF, jnp.int32)

        def adj_body(r, carry):
            sl = pl.ds(r * L, L)
            v = big_s[sl]
            big_d[sl] = lax.shift_right_logical(v, 14)
            big_s[sl] = (v & maskv) + off_s
            return carry

        lax.fori_loop(0, ept // L, adj_body, 0)

        def fill_da(b, j):
            for g in range(CHUNK // L):
                sl = pl.ds(g * L, L)
                da_b[b][sl] = big_d[pl.ds(j * CHUNK + g * L, L)] + off_d

        def fill_db(b, j):
            for g in range(CHUNK // L):
                sl = pl.ds(g * L, L)
                db_b[b][sl] = big_d[pl.ds(j * CHUNK + g * L, L)]

        def issue_gathers(b, j):
            fill_da(b, j)
            pltpu.async_copy(tab_hbm.at[big_s.at[pl.ds(j * CHUNK, CHUNK)]],
                             xl_b[b], gs_b[b])
            pltpu.async_copy(tab_hbm.at[da_b[b]], xr_b[b], gd_b[b])

        def wait_gathers(b):
            pltpu.make_async_copy(tab_hbm.at[da_b[b]], xl_b[b], gs_b[b]).wait()
            pltpu.make_async_copy(tab_hbm.at[da_b[b]], xr_b[b], gd_b[b]).wait()

        def wait_scatters(b):
            pltpu.make_async_copy(msg_b[b], numer_sh.at[db_b[b]], sm_b[b]).wait()
            pltpu.make_async_copy(ex_b[b], denom_sh.at[db_b[b]], se_b[b]).wait()

        def compute_chunk(b):
            xlb, xrb, msgb, exb = xl_b[b], xr_b[b], msg_b[b], ex_b[b]

            def edge_body(e, ecarry):
                als, ts = [], []
                for k in range(HL):
                    a = xlb[e, pl.ds(k * L, L)]
                    bb = xrb[e, pl.ds(k * L, L)]
                    s = a + bb
                    s = jnp.maximum(s, zeros16) + negv * jnp.minimum(s, zeros16)
                    t = s * att_vecs[k]
                    for p in perms:
                        t = t + t.at[p].get(mode="promise_in_bounds")
                    als.append(a)
                    ts.append(t)
                z = (ts[0] * hot_vecs[0] + ts[1] * hot_vecs[1]
                     + ts[2] * hot_vecs[2] + ts[3] * hot_vecs[3])
                ez = jnp.exp(z)
                exb[e, :] = ez * lo4_vec
                for k in range(HL):
                    sk = ez.at[splat_pat[k]].get(mode="promise_in_bounds")
                    msgb[e, pl.ds(k * L, L)] = sk * als[k]
                return ecarry

            lax.fori_loop(0, CHUNK, edge_body, 0)

        issue_gathers(0, 0)
        issue_gathers(1, jnp.int32(1))

        def pipe_body(i, carry):
            for b in range(2):
                j = 2 * i + b
                wait_gathers(b)

                @pl.when(i >= 1)
                def _():
                    wait_scatters(b)

                compute_chunk(b)
                fill_db(b, j)
                pltpu.async_copy(msg_b[b], numer_sh.at[db_b[b]], sm_b[b],
                                 add=True)
                pltpu.async_copy(ex_b[b], denom_sh.at[db_b[b]], se_b[b],
                                 add=True)
                jn = jnp.minimum(j + 2, cpt - 1)
                issue_gathers(b, jn)
            return carry

        lax.fori_loop(0, cpt // 2, pipe_body, 0)
        for b in range(2):
            wait_gathers(b)
            wait_scatters(b)

        plsc.subcore_barrier()

        bias_vecs = [bias_buf[k, :] for k in range(HL)]
        epsv = jnp.full((L,), 1e-16, jnp.float32)
        idxk = [jnp.full((L,), k, jnp.int32) for k in range(HL)]
        done = 0
        for sz in pieces:
            r0 = row0 + done
            pltpu.sync_copy(numer_sh.at[pl.ds(r0, sz)], xl0.at[pl.ds(0, sz)])
            pltpu.sync_copy(denom_sh.at[pl.ds(r0, sz)], exb0.at[pl.ds(0, sz)])

            def div_body(r, carry):
                den_row = exb0[r, :]
                for k in range(HL):
                    dk = den_row.at[idxk[k]].get(mode="promise_in_bounds")
                    sl = pl.ds(k * L, L)
                    msg0[r, sl] = xl0[r, sl] / (dk + epsv) + bias_vecs[k]
                return carry

            lax.fori_loop(0, sz, div_body, 0)
            pltpu.sync_copy(msg0.at[pl.ds(0, sz)],
                            out_hbm.at[cid, pl.ds(r0, sz)])
            done += sz

    return edge_kernel


def kernel(x, edge_index, W_l, W_r, att, bias):
    E2 = edge_index.shape[1]
    etot = E2 + N
    loop = jnp.arange(N, dtype=edge_index.dtype)
    src = jnp.concatenate([edge_index[0], loop])
    dst = jnp.concatenate([edge_index[1], loop])

    edges_per_tile = -(-etot // (NS * 2 * CHUNK)) * 2 * CHUNK
    cpt = edges_per_tile // CHUNK
    epad = edges_per_tile * NS
    src = jnp.pad(src, (0, epad - etot), constant_values=N)
    dst = jnp.pad(dst, (0, epad - etot), constant_values=N)
    edges_packed = src | (dst << 14)

    xpad = jnp.pad(x, ((0, NXL - N), (0, 0)))
    wcat = jnp.stack([W_l[:, :DH], W_l[:, DH:], W_r[:, :DH], W_r[:, DH:]])
    tab = _project(xpad, wcat)

    att_g = att.reshape(NC, HL, C)
    bias_g = bias.astype(jnp.float32).reshape(NC, HL, C)
    hot = jnp.concatenate([
        jnp.eye(HL, L, dtype=jnp.float32),
        (jnp.arange(L) < HL).astype(jnp.float32)[None, :],
    ], axis=0)
    offs = jnp.stack([
        jnp.full((2, L), 0 * NXL, jnp.int32) + jnp.array([[0], [2 * NXL]], jnp.int32),
        jnp.full((2, L), 1 * NXL, jnp.int32) + jnp.array([[0], [2 * NXL]], jnp.int32),
    ])

    edge_kernel = _make_edge_kernel(cpt)
    out_halves = edge_kernel(tab, edges_packed, att_g, hot, bias_g, offs)

    return out_halves.transpose(1, 0, 2).reshape(NPAD, D)[:N]


# speedup vs baseline: 1.5303x; 1.4393x over previous
import functools

import jax
import jax.numpy as jnp
from jax import lax
from jax.experimental import pallas as pl
from jax.experimental.pallas import tpu as pltpu
from jax.experimental.pallas import tpu_sc as plsc

N = 10000
D = 128
DH = 64
H = 8
HL = 4
C = 16
NEG = 0.2
NC, NS, L = 2, 16, 16
CHUNK = 64
NPAD = 10016
NXL = 10016
ROWS_PER_TILE = NPAD // NS


def _proj_body(x_ref, w_ref, out_ref):
    x = x_ref[...]
    w = w_ref[0]
    out_ref[0] = jnp.dot(x, w, preferred_element_type=jnp.float32)


def _project(xpad, wcat):
    blk = 2504
    grid_i = NXL // blk
    out = pl.pallas_call(
        _proj_body,
        grid=(4, grid_i),
        in_specs=[
            pl.BlockSpec((blk, D), lambda q, i: (i, 0)),
            pl.BlockSpec((1, D, DH), lambda q, i: (q, 0, 0)),
        ],
        out_specs=pl.BlockSpec((1, blk, DH), lambda q, i: (q, i, 0)),
        out_shape=jax.ShapeDtypeStruct((4, NXL, DH), jnp.float32),
    )(xpad, wcat)
    return out.reshape(4 * NXL, DH)


def _make_edge_kernel(cpt):
    mesh = plsc.VectorSubcoreMesh(
        core_axis_name="c", subcore_axis_name="s", num_cores=NC, num_subcores=NS
    )

    @functools.partial(
        pl.kernel,
        mesh=mesh,
        compiler_params=pltpu.CompilerParams(use_tc_tiling_on_sc=False),
        out_type=jax.ShapeDtypeStruct((NC, NPAD, DH), jnp.float32),
        scratch_types=[
            pltpu.VMEM((cpt * CHUNK,), jnp.int32),
            pltpu.VMEM((cpt * CHUNK,), jnp.int32),
            pltpu.VMEM((CHUNK,), jnp.int32),
            pltpu.VMEM((CHUNK,), jnp.int32),
            pltpu.VMEM((CHUNK, DH), jnp.float32),
            pltpu.VMEM((CHUNK, DH), jnp.float32),
            pltpu.VMEM((CHUNK, DH), jnp.float32),
            pltpu.VMEM((CHUNK, DH), jnp.float32),
            pltpu.VMEM((CHUNK, DH), jnp.float32),
            pltpu.VMEM((CHUNK, DH), jnp.float32),
            pltpu.VMEM((CHUNK, L), jnp.float32),
            pltpu.VMEM((CHUNK, L), jnp.float32),
            pltpu.VMEM((CHUNK,), jnp.int32),
            pltpu.VMEM((CHUNK,), jnp.int32),
            pltpu.VMEM((HL, L), jnp.float32),
            pltpu.VMEM((HL, L), jnp.float32),
            pltpu.VMEM((HL, L), jnp.float32),
            pltpu.VMEM((2, L), jnp.int32),
            pltpu.VMEM_SHARED((NPAD, DH), jnp.float32),
            pltpu.VMEM_SHARED((NPAD, L), jnp.float32),
            pltpu.SemaphoreType.DMA,
            pltpu.SemaphoreType.DMA,
            pltpu.SemaphoreType.DMA,
            pltpu.SemaphoreType.DMA,
            pltpu.SemaphoreType.DMA,
            pltpu.SemaphoreType.DMA,
            pltpu.SemaphoreType.DMA,
            pltpu.SemaphoreType.DMA,
        ],
    )
    def edge_kernel(tab_hbm, edges_hbm, att_hbm, hot_hbm, bias_hbm,
                    offs_hbm, out_hbm,
                    big_s, big_d, db0, db1, xl0, xl1, xr0, xr1, msg0, msg1,
                    exb0, exb1, da0, da1, att_buf, hot_buf, bias_buf, off_buf,
                    numer_sh, denom_sh,
                    gs0, gs1, gd0, gd1, sm0, sm1, se0, se1):
        cid = lax.axis_index("c")
        sid = lax.axis_index("s")
        row0 = sid * ROWS_PER_TILE

        xl_b = (xl0, xl1)
        xr_b = (xr0, xr1)
        msg_b = (msg0, msg1)
        ex_b = (exb0, exb1)
        da_b = (da0, da1)
        db_b = (db0, db1)
        gs_b = (gs0, gs1)
        gd_b = (gd0, gd1)
        sm_b = (sm0, sm1)
        se_b = (se0, se1)

        zeros16 = jnp.zeros((L,), jnp.float32)
        negv = jnp.full((L,), NEG, jnp.float32)
        lane = lax.iota(jnp.int32, L)
        perms = [lane ^ (1 << p) for p in range(4)]

        def zbody(r, carry):
            for g in range(DH // L):
                msg0[r, pl.ds(g * L, L)] = zeros16
            exb0[r, :] = zeros16
            return carry

        lax.fori_loop(0, CHUNK, zbody, 0)

        npc = ROWS_PER_TILE // CHUNK
        pieces = [CHUNK] * npc
        if ROWS_PER_TILE % CHUNK:
            pieces.append(ROWS_PER_TILE % CHUNK)
        zdone = 0
        for sz in pieces:
            pltpu.sync_copy(msg0.at[pl.ds(0, sz)],
                            numer_sh.at[pl.ds(row0 + zdone, sz)])
            pltpu.sync_copy(exb0.at[pl.ds(0, sz)],
                            denom_sh.at[pl.ds(row0 + zdone, sz)])
            zdone += sz
        plsc.subcore_barrier()

        pltpu.sync_copy(att_hbm.at[cid], att_buf)
        pltpu.sync_copy(hot_hbm, hot_buf)
        pltpu.sync_copy(bias_hbm.at[cid], bias_buf)
        pltpu.sync_copy(offs_hbm.at[cid], off_buf)
        att_vecs = [att_buf[k, :] for k in range(HL)]
        hot_vecs = [hot_buf[k, :] for k in range(HL)]
        off_s = off_buf[0, :]
        off_d = off_buf[1, :]

        ept = cpt * CHUNK
        pltpu.sync_copy(edges_hbm.at[pl.ds(sid * ept, ept)], big_s)
        maskv = jnp.full((L,), 0x3FFF, jnp.int32)

        def adj_body(r, carry):
            sl = pl.ds(r * L, L)
            v = big_s[sl]
            big_d[sl] = lax.shift_right_logical(v, 14)
            big_s[sl] = (v & maskv) + off_s
            return carry

        lax.fori_loop(0, ept // L, adj_body, 0)

        def fill_da(b, j):
            for g in range(CHUNK // L):
                sl = pl.ds(g * L, L)
                da_b[b][sl] = big_d[pl.ds(j * CHUNK + g * L, L)] + off_d

        def fill_db(b, j):
            for g in range(CHUNK // L):
                sl = pl.ds(g * L, L)
                db_b[b][sl] = big_d[pl.ds(j * CHUNK + g * L, L)]

        def issue_gathers(b, j):
            fill_da(b, j)
            pltpu.async_copy(tab_hbm.at[big_s.at[pl.ds(j * CHUNK, CHUNK)]],
                             xl_b[b], gs_b[b])
            pltpu.async_copy(tab_hbm.at[da_b[b]], xr_b[b], gd_b[b])

        def wait_gathers(b):
            pltpu.make_async_copy(tab_hbm.at[da_b[b]], xl_b[b], gs_b[b]).wait()
            pltpu.make_async_copy(tab_hbm.at[da_b[b]], xr_b[b], gd_b[b]).wait()

        def wait_scatters(b):
            pltpu.make_async_copy(msg_b[b], numer_sh.at[db_b[b]], sm_b[b]).wait()
            pltpu.make_async_copy(ex_b[b], denom_sh.at[db_b[b]], se_b[b]).wait()

        def compute_chunk(b):
            xlb, xrb, msgb, exb = xl_b[b], xr_b[b], msg_b[b], ex_b[b]

            def edge_body(i, ecarry):
                ee = (2 * i, 2 * i + 1)
                exrow = [jnp.zeros((L,), jnp.float32)] * 2
                for k in range(HL):
                    for u, e in enumerate(ee):
                        a = xlb[e, pl.ds(k * L, L)]
                        bb = xrb[e, pl.ds(k * L, L)]
                        s = a + bb
                        s = (jnp.maximum(s, zeros16)
                             + negv * jnp.minimum(s, zeros16))
                        t = s * att_vecs[k]
                        for p in perms:
                            t = t + t.at[p].get(mode="promise_in_bounds")
                        ex = jnp.exp(t)
                        msgb[e, pl.ds(k * L, L)] = ex * a
                        exrow[u] = exrow[u] + ex * hot_vecs[k]
                for u, e in enumerate(ee):
                    exb[e, :] = exrow[u]
                return ecarry

            lax.fori_loop(0, CHUNK // 2, edge_body, 0)

        issue_gathers(0, 0)
        issue_gathers(1, jnp.int32(1))

        def pipe_body(i, carry):
            for b in range(2):
                j = 2 * i + b
                wait_gathers(b)

                @pl.when(i >= 1)
                def _():
                    wait_scatters(b)

                compute_chunk(b)
                fill_db(b, j)
                pltpu.async_copy(msg_b[b], numer_sh.at[db_b[b]], sm_b[b],
                                 add=True)
                pltpu.async_copy(ex_b[b], denom_sh.at[db_b[b]], se_b[b],
                                 add=True)
                jn = jnp.minimum(j + 2, cpt - 1)
                issue_gathers(b, jn)
            return carry

        lax.fori_loop(0, cpt // 2, pipe_body, 0)
        for b in range(2):
            wait_gathers(b)
            wait_scatters(b)

        plsc.subcore_barrier()

        bias_vecs = [bias_buf[k, :] for k in range(HL)]
        epsv = jnp.full((L,), 1e-16, jnp.float32)
        idxk = [jnp.full((L,), k, jnp.int32) for k in range(HL)]
        done = 0
        for sz in pieces:
            r0 = row0 + done
            pltpu.sync_copy(numer_sh.at[pl.ds(r0, sz)], xl0.at[pl.ds(0, sz)])
            pltpu.sync_copy(denom_sh.at[pl.ds(r0, sz)], exb0.at[pl.ds(0, sz)])

            def div_body(r, carry):
                den_row = exb0[r, :]
                for k in range(HL):
                    dk = den_row.at[idxk[k]].get(mode="promise_in_bounds")
                    sl = pl.ds(k * L, L)
                    msg0[r, sl] = xl0[r, sl] / (dk + epsv) + bias_vecs[k]
                return carry

            lax.fori_loop(0, sz, div_body, 0)
            pltpu.sync_copy(msg0.at[pl.ds(0, sz)],
                            out_hbm.at[cid, pl.ds(r0, sz)])
            done += sz

    return edge_kernel


def kernel(x, edge_index, W_l, W_r, att, bias):
    E2 = edge_index.shape[1]
    etot = E2 + N
    loop = jnp.arange(N, dtype=edge_index.dtype)
    src = jnp.concatenate([edge_index[0], loop])
    dst = jnp.concatenate([edge_index[1], loop])

    edges_per_tile = -(-etot // (NS * 2 * CHUNK)) * 2 * CHUNK
    cpt = edges_per_tile // CHUNK
    epad = edges_per_tile * NS
    src = jnp.pad(src, (0, epad - etot), constant_values=N)
    dst = jnp.pad(dst, (0, epad - etot), constant_values=N)
    edges_packed = src | (dst << 14)

    xpad = jnp.pad(x, ((0, NXL - N), (0, 0)))
    wcat = jnp.stack([W_l[:, :DH], W_l[:, DH:], W_r[:, :DH], W_r[:, DH:]])
    tab = _project(xpad, wcat)

    att_g = att.reshape(NC, HL, C)
    bias_g = bias.astype(jnp.float32).reshape(NC, HL, C)
    hot = jnp.eye(HL, L, dtype=jnp.float32)
    offs = jnp.stack([
        jnp.full((2, L), 0 * NXL, jnp.int32) + jnp.array([[0], [2 * NXL]], jnp.int32),
        jnp.full((2, L), 1 * NXL, jnp.int32) + jnp.array([[0], [2 * NXL]], jnp.int32),
    ])

    edge_kernel = _make_edge_kernel(cpt)
    out_halves = edge_kernel(tab, edges_packed, att_g, hot, bias_g, offs)

    return out_halves.transpose(1, 0, 2).reshape(NPAD, D)[:N]


# speedup vs baseline: 1.6636x; 1.0871x over previous
import functools

import jax
import jax.numpy as jnp
from jax import lax
from jax.experimental import pallas as pl
from jax.experimental.pallas import tpu as pltpu
from jax.experimental.pallas import tpu_sc as plsc

N = 10000
D = 128
DH = 64
H = 8
HL = 4
C = 16
NEG = 0.2
NC, NS, L = 2, 16, 16
CHUNK = 64
NPAD = 10016
NXL = 10016
ROWS_PER_TILE = NPAD // NS


def _proj_body(x_ref, w_ref, out_ref):
    x = x_ref[...]
    w = w_ref[0]
    out_ref[0] = jnp.dot(x, w, preferred_element_type=jnp.float32)


def _project(xpad, wcat):
    blk = 2504
    grid_i = NXL // blk
    out = pl.pallas_call(
        _proj_body,
        grid=(4, grid_i),
        in_specs=[
            pl.BlockSpec((blk, D), lambda q, i: (i, 0)),
            pl.BlockSpec((1, D, DH), lambda q, i: (q, 0, 0)),
        ],
        out_specs=pl.BlockSpec((1, blk, DH), lambda q, i: (q, i, 0)),
        out_shape=jax.ShapeDtypeStruct((4, NXL, DH), jnp.float32),
    )(xpad, wcat)
    return out.reshape(4 * NXL, DH)


def _make_edge_kernel(cpt):
    mesh = plsc.VectorSubcoreMesh(
        core_axis_name="c", subcore_axis_name="s", num_cores=NC, num_subcores=NS
    )

    @functools.partial(
        pl.kernel,
        mesh=mesh,
        compiler_params=pltpu.CompilerParams(use_tc_tiling_on_sc=False),
        out_type=jax.ShapeDtypeStruct((NC, NPAD, DH), jnp.float32),
        scratch_types=[
            pltpu.VMEM((cpt * CHUNK,), jnp.int32),
            pltpu.VMEM((cpt * CHUNK,), jnp.int32),
            pltpu.VMEM((CHUNK,), jnp.int32),
            pltpu.VMEM((CHUNK,), jnp.int32),
            pltpu.VMEM((CHUNK, DH), jnp.float32),
            pltpu.VMEM((CHUNK, DH), jnp.float32),
            pltpu.VMEM((CHUNK, DH), jnp.float32),
            pltpu.VMEM((CHUNK, DH), jnp.float32),
            pltpu.VMEM((CHUNK, DH), jnp.float32),
            pltpu.VMEM((CHUNK, DH), jnp.float32),
            pltpu.VMEM((CHUNK, L), jnp.float32),
            pltpu.VMEM((CHUNK, L), jnp.float32),
            pltpu.VMEM((CHUNK,), jnp.int32),
            pltpu.VMEM((CHUNK,), jnp.int32),
            pltpu.VMEM((HL, L), jnp.float32),
            pltpu.VMEM((HL, L), jnp.float32),
            pltpu.VMEM((HL, L), jnp.float32),
            pltpu.VMEM((2, L), jnp.int32),
            pltpu.VMEM_SHARED((NPAD, DH), jnp.float32),
            pltpu.VMEM_SHARED((NPAD, L), jnp.float32),
            pltpu.SemaphoreType.DMA,
            pltpu.SemaphoreType.DMA,
            pltpu.SemaphoreType.DMA,
            pltpu.SemaphoreType.DMA,
            pltpu.SemaphoreType.DMA,
            pltpu.SemaphoreType.DMA,
            pltpu.SemaphoreType.DMA,
            pltpu.SemaphoreType.DMA,
        ],
    )
    def edge_kernel(tab_hbm, edges_hbm, att_hbm, hot_hbm, bias_hbm,
                    offs_hbm, out_hbm,
                    big_s, big_d, db0, db1, xl0, xl1, xr0, xr1, msg0, msg1,
                    exb0, exb1, da0, da1, att_buf, hot_buf, bias_buf, off_buf,
                    numer_sh, denom_sh,
                    gs0, gs1, gd0, gd1, sm0, sm1, se0, se1):
        cid = lax.axis_index("c")
        sid = lax.axis_index("s")
        row0 = sid * ROWS_PER_TILE

        xl_b = (xl0, xl1)
        xr_b = (xr0, xr1)
        msg_b = (msg0, msg1)
        ex_b = (exb0, exb1)
        da_b = (da0, da1)
        db_b = (db0, db1)
        gs_b = (gs0, gs1)
        gd_b = (gd0, gd1)
        sm_b = (sm0, sm1)
        se_b = (se0, se1)

        zeros16 = jnp.zeros((L,), jnp.float32)
        negv = jnp.full((L,), NEG, jnp.float32)
        lane = lax.iota(jnp.int32, L)
        perms = [lane ^ (1 << p) for p in range(4)]

        def zbody(r, carry):
            for g in range(DH // L):
                msg0[r, pl.ds(g * L, L)] = zeros16
            exb0[r, :] = zeros16
            return carry

        lax.fori_loop(0, CHUNK, zbody, 0)

        npc = ROWS_PER_TILE // CHUNK
        pieces = [CHUNK] * npc
        if ROWS_PER_TILE % CHUNK:
            pieces.append(ROWS_PER_TILE % CHUNK)
        zdone = 0
        for sz in pieces:
            pltpu.sync_copy(msg0.at[pl.ds(0, sz)],
                            numer_sh.at[pl.ds(row0 + zdone, sz)])
            pltpu.sync_copy(exb0.at[pl.ds(0, sz)],
                            denom_sh.at[pl.ds(row0 + zdone, sz)])
            zdone += sz
        plsc.subcore_barrier()

        pltpu.sync_copy(att_hbm.at[cid], att_buf)
        pltpu.sync_copy(hot_hbm, hot_buf)
        pltpu.sync_copy(bias_hbm.at[cid], bias_buf)
        pltpu.sync_copy(offs_hbm.at[cid], off_buf)
        att_vecs = [att_buf[k, :] for k in range(HL)]
        hot_vecs = [hot_buf[k, :] for k in range(HL)]
        off_s = off_buf[0, :]
        off_d = off_buf[1, :]

        ept = cpt * CHUNK
        pltpu.sync_copy(edges_hbm.at[pl.ds(sid * ept, ept)], big_s)
        maskv = jnp.full((L,), 0x3FFF, jnp.int32)

        def adj_body(r, carry):
            sl = pl.ds(r * L, L)
            v = big_s[sl]
            big_d[sl] = lax.shift_right_logical(v, 14)
            big_s[sl] = (v & maskv) + off_s
            return carry

        lax.fori_loop(0, ept // L, adj_body, 0)

        def fill_da(b, j):
            for g in range(CHUNK // L):
                sl = pl.ds(g * L, L)
                da_b[b][sl] = big_d[pl.ds(j * CHUNK + g * L, L)] + off_d

        def fill_db(b, j):
            for g in range(CHUNK // L):
                sl = pl.ds(g * L, L)
                db_b[b][sl] = big_d[pl.ds(j * CHUNK + g * L, L)]

        def issue_gathers(b, j):
            fill_da(b, j)
            pltpu.async_copy(tab_hbm.at[big_s.at[pl.ds(j * CHUNK, CHUNK)]],
                             xl_b[b], gs_b[b])
            pltpu.async_copy(tab_hbm.at[da_b[b]], xr_b[b], gd_b[b])

        def wait_gathers(b):
            pltpu.make_async_copy(tab_hbm.at[da_b[b]], xl_b[b], gs_b[b]).wait()
            pltpu.make_async_copy(tab_hbm.at[da_b[b]], xr_b[b], gd_b[b]).wait()

        def wait_scatters(b):
            pltpu.make_async_copy(msg_b[b], numer_sh.at[db_b[b]], sm_b[b]).wait()
            pltpu.make_async_copy(ex_b[b], denom_sh.at[db_b[b]], se_b[b]).wait()

        def compute_chunk(b):
            xlb, xrb, msgb, exb = xl_b[b], xr_b[b], msg_b[b], ex_b[b]

            def edge_body(i, ecarry):
                ee = (4 * i, 4 * i + 1, 4 * i + 2, 4 * i + 3)
                exrow = [jnp.zeros((L,), jnp.float32)] * 4
                for k in range(HL):
                    for u, e in enumerate(ee):
                        a = xlb[e, pl.ds(k * L, L)]
                        bb = xrb[e, pl.ds(k * L, L)]
                        s = a + bb
                        s = (jnp.maximum(s, zeros16)
                             + negv * jnp.minimum(s, zeros16))
                        t = s * att_vecs[k]
                        for p in perms:
                            t = t + t.at[p].get(mode="promise_in_bounds")
                        ex = jnp.exp(t)
                        msgb[e, pl.ds(k * L, L)] = ex * a
                        exrow[u] = exrow[u] + ex * hot_vecs[k]
                for u, e in enumerate(ee):
                    exb[e, :] = exrow[u]
                return ecarry

            lax.fori_loop(0, CHUNK // 4, edge_body, 0)

        issue_gathers(0, 0)
        issue_gathers(1, jnp.int32(1))

        def pipe_body(i, carry):
            for b in range(2):
                j = 2 * i + b
                wait_gathers(b)

                @pl.when(i >= 1)
                def _():
                    wait_scatters(b)

                compute_chunk(b)
                fill_db(b, j)
                pltpu.async_copy(msg_b[b], numer_sh.at[db_b[b]], sm_b[b],
                                 add=True)
                pltpu.async_copy(ex_b[b], denom_sh.at[db_b[b]], se_b[b],
                                 add=True)
                jn = jnp.minimum(j + 2, cpt - 1)
                issue_gathers(b, jn)
            return carry

        lax.fori_loop(0, cpt // 2, pipe_body, 0)
        for b in range(2):
            wait_gathers(b)
            wait_scatters(b)

        plsc.subcore_barrier()

        bias_vecs = [bias_buf[k, :] for k in range(HL)]
        epsv = jnp.full((L,), 1e-16, jnp.float32)
        idxk = [jnp.full((L,), k, jnp.int32) for k in range(HL)]
        done = 0
        for sz in pieces:
            r0 = row0 + done
            pltpu.sync_copy(numer_sh.at[pl.ds(r0, sz)], xl0.at[pl.ds(0, sz)])
            pltpu.sync_copy(denom_sh.at[pl.ds(r0, sz)], exb0.at[pl.ds(0, sz)])

            def div_body(r, carry):
                den_row = exb0[r, :]
                for k in range(HL):
                    dk = den_row.at[idxk[k]].get(mode="promise_in_bounds")
                    sl = pl.ds(k * L, L)
                    msg0[r, sl] = xl0[r, sl] / (dk + epsv) + bias_vecs[k]
                return carry

            lax.fori_loop(0, sz, div_body, 0)
            pltpu.sync_copy(msg0.at[pl.ds(0, sz)],
                            out_hbm.at[cid, pl.ds(r0, sz)])
            done += sz

    return edge_kernel


def kernel(x, edge_index, W_l, W_r, att, bias):
    E2 = edge_index.shape[1]
    etot = E2 + N
    loop = jnp.arange(N, dtype=edge_index.dtype)
    src = jnp.concatenate([edge_index[0], loop])
    dst = jnp.concatenate([edge_index[1], loop])

    edges_per_tile = -(-etot // (NS * 2 * CHUNK)) * 2 * CHUNK
    cpt = edges_per_tile // CHUNK
    epad = edges_per_tile * NS
    src = jnp.pad(src, (0, epad - etot), constant_values=N)
    dst = jnp.pad(dst, (0, epad - etot), constant_values=N)
    edges_packed = src | (dst << 14)

    xpad = jnp.pad(x, ((0, NXL - N), (0, 0)))
    wcat = jnp.stack([W_l[:, :DH], W_l[:, DH:], W_r[:, :DH], W_r[:, DH:]])
    tab = _project(xpad, wcat)

    att_g = att.reshape(NC, HL, C)
    bias_g = bias.astype(jnp.float32).reshape(NC, HL, C)
    hot = jnp.eye(HL, L, dtype=jnp.float32)
    offs = jnp.stack([
        jnp.full((2, L), 0 * NXL, jnp.int32) + jnp.array([[0], [2 * NXL]], jnp.int32),
        jnp.full((2, L), 1 * NXL, jnp.int32) + jnp.array([[0], [2 * NXL]], jnp.int32),
    ])

    edge_kernel = _make_edge_kernel(cpt)
    out_halves = edge_kernel(tab, edges_packed, att_g, hot, bias_g, offs)

    return out_halves.transpose(1, 0, 2).reshape(NPAD, D)[:N]


# speedup vs baseline: 1.7379x; 1.0447x over previous
import functools

import jax
import jax.numpy as jnp
from jax import lax
from jax.experimental import pallas as pl
from jax.experimental.pallas import tpu as pltpu
from jax.experimental.pallas import tpu_sc as plsc

N = 10000
D = 128
DH = 64
H = 8
HL = 4
C = 16
NEG = 0.2
NC, NS, L = 2, 16, 16
CHUNK = 64
NPAD = 10016
NXL = 10016
ROWS_PER_TILE = NPAD // NS


def _proj_body(x_ref, w_ref, out_ref):
    x = x_ref[...]
    w = w_ref[0]
    out_ref[0] = jnp.dot(x, w, preferred_element_type=jnp.float32)


def _project(xpad, wcat):
    blk = 2504
    grid_i = NXL // blk
    out = pl.pallas_call(
        _proj_body,
        grid=(4, grid_i),
        in_specs=[
            pl.BlockSpec((blk, D), lambda q, i: (i, 0)),
            pl.BlockSpec((1, D, DH), lambda q, i: (q, 0, 0)),
        ],
        out_specs=pl.BlockSpec((1, blk, DH), lambda q, i: (q, i, 0)),
        out_shape=jax.ShapeDtypeStruct((4, NXL, DH), jnp.float32),
    )(xpad, wcat)
    return out.reshape(4 * NXL, DH)


def _make_edge_kernel(cpt):
    mesh = plsc.VectorSubcoreMesh(
        core_axis_name="c", subcore_axis_name="s", num_cores=NC, num_subcores=NS
    )

    @functools.partial(
        pl.kernel,
        mesh=mesh,
        compiler_params=pltpu.CompilerParams(use_tc_tiling_on_sc=False),
        out_type=jax.ShapeDtypeStruct((NC, NPAD, DH), jnp.float32),
        scratch_types=[
            pltpu.VMEM((cpt * CHUNK,), jnp.int32),
            pltpu.VMEM((cpt * CHUNK,), jnp.int32),
            pltpu.VMEM((CHUNK,), jnp.int32),
            pltpu.VMEM((CHUNK,), jnp.int32),
            pltpu.VMEM((CHUNK, DH), jnp.float32),
            pltpu.VMEM((CHUNK, DH), jnp.float32),
            pltpu.VMEM((CHUNK, DH), jnp.float32),
            pltpu.VMEM((CHUNK, DH), jnp.float32),
            pltpu.VMEM((CHUNK, DH), jnp.float32),
            pltpu.VMEM((CHUNK, DH), jnp.float32),
            pltpu.VMEM((CHUNK, L), jnp.float32),
            pltpu.VMEM((CHUNK, L), jnp.float32),
            pltpu.VMEM((CHUNK,), jnp.int32),
            pltpu.VMEM((CHUNK,), jnp.int32),
            pltpu.VMEM((HL, L), jnp.float32),
            pltpu.VMEM((HL, L), jnp.float32),
            pltpu.VMEM((HL, L), jnp.float32),
            pltpu.VMEM((2, L), jnp.int32),
            pltpu.VMEM_SHARED((NPAD, DH), jnp.float32),
            pltpu.VMEM_SHARED((NPAD, L), jnp.float32),
            pltpu.SemaphoreType.DMA,
            pltpu.SemaphoreType.DMA,
            pltpu.SemaphoreType.DMA,
            pltpu.SemaphoreType.DMA,
            pltpu.SemaphoreType.DMA,
            pltpu.SemaphoreType.DMA,
            pltpu.SemaphoreType.DMA,
            pltpu.SemaphoreType.DMA,
        ],
    )
    def edge_kernel(tab_hbm, edges_hbm, att_hbm, hot_hbm, bias_hbm,
                    offs_hbm, out_hbm,
                    big_s, big_d, db0, db1, xl0, xl1, xr0, xr1, msg0, msg1,
                    exb0, exb1, da0, da1, att_buf, hot_buf, bias_buf, off_buf,
                    numer_sh, denom_sh,
                    gs0, gs1, gd0, gd1, sm0, sm1, se0, se1):
        cid = lax.axis_index("c")
        sid = lax.axis_index("s")
        row0 = sid * ROWS_PER_TILE

        xl_b = (xl0, xl1)
        xr_b = (xr0, xr1)
        msg_b = (msg0, msg1)
        ex_b = (exb0, exb1)
        da_b = (da0, da1)
        db_b = (db0, db1)
        gs_b = (gs0, gs1)
        gd_b = (gd0, gd1)
        sm_b = (sm0, sm1)
        se_b = (se0, se1)

        zeros16 = jnp.zeros((L,), jnp.float32)
        negv = jnp.full((L,), NEG, jnp.float32)
        lane = lax.iota(jnp.int32, L)
        perms = [lane ^ (1 << p) for p in range(4)]

        def zbody(r, carry):
            for g in range(DH // L):
                msg0[r, pl.ds(g * L, L)] = zeros16
            exb0[r, :] = zeros16
            return carry

        lax.fori_loop(0, CHUNK, zbody, 0)

        npc = ROWS_PER_TILE // CHUNK
        pieces = [CHUNK] * npc
        if ROWS_PER_TILE % CHUNK:
            pieces.append(ROWS_PER_TILE % CHUNK)
        zdone = 0
        for sz in pieces:
            pltpu.sync_copy(msg0.at[pl.ds(0, sz)],
                            numer_sh.at[pl.ds(row0 + zdone, sz)])
            pltpu.sync_copy(exb0.at[pl.ds(0, sz)],
                            denom_sh.at[pl.ds(row0 + zdone, sz)])
            zdone += sz
        plsc.subcore_barrier()

        pltpu.sync_copy(att_hbm.at[cid], att_buf)
        pltpu.sync_copy(hot_hbm, hot_buf)
        pltpu.sync_copy(bias_hbm.at[cid], bias_buf)
        pltpu.sync_copy(offs_hbm.at[cid], off_buf)
        att_vecs = [att_buf[k, :] for k in range(HL)]
        hot_vecs = [hot_buf[k, :] for k in range(HL)]
        off_s = off_buf[0, :]
        off_d = off_buf[1, :]

        ept = cpt * CHUNK
        pltpu.sync_copy(edges_hbm.at[pl.ds(sid * ept, ept)], big_s)
        maskv = jnp.full((L,), 0x3FFF, jnp.int32)

        def adj_body(r, carry):
            sl = pl.ds(r * L, L)
            v = big_s[sl]
            big_d[sl] = lax.shift_right_logical(v, 14)
            big_s[sl] = (v & maskv) + off_s
            return carry

        lax.fori_loop(0, ept // L, adj_body, 0)

        def fill_da(b, j):
            for g in range(CHUNK // L):
                sl = pl.ds(g * L, L)
                da_b[b][sl] = big_d[pl.ds(j * CHUNK + g * L, L)] + off_d

        def fill_db(b, j):
            for g in range(CHUNK // L):
                sl = pl.ds(g * L, L)
                db_b[b][sl] = big_d[pl.ds(j * CHUNK + g * L, L)]

        def issue_gathers(b, j):
            fill_da(b, j)
            pltpu.async_copy(tab_hbm.at[big_s.at[pl.ds(j * CHUNK, CHUNK)]],
                             xl_b[b], gs_b[b])
            pltpu.async_copy(tab_hbm.at[da_b[b]], xr_b[b], gd_b[b])

        def wait_gathers(b):
            pltpu.make_async_copy(tab_hbm.at[da_b[b]], xl_b[b], gs_b[b]).wait()
            pltpu.make_async_copy(tab_hbm.at[da_b[b]], xr_b[b], gd_b[b]).wait()

        def wait_scatters(b):
            pltpu.make_async_copy(msg_b[b], numer_sh.at[db_b[b]], sm_b[b]).wait()
            pltpu.make_async_copy(ex_b[b], denom_sh.at[db_b[b]], se_b[b]).wait()

        def compute_chunk(b):
            xlb, xrb, msgb, exb = xl_b[b], xr_b[b], msg_b[b], ex_b[b]

            def edge_body(i, ecarry):
                ee = tuple(8 * i + u for u in range(8))
                exrow = [jnp.zeros((L,), jnp.float32)] * 8
                for k in range(HL):
                    for u, e in enumerate(ee):
                        a = xlb[e, pl.ds(k * L, L)]
                        bb = xrb[e, pl.ds(k * L, L)]
                        s = a + bb
                        s = (jnp.maximum(s, zeros16)
                             + negv * jnp.minimum(s, zeros16))
                        t = s * att_vecs[k]
                        for p in perms:
                            t = t + t.at[p].get(mode="promise_in_bounds")
                        ex = jnp.exp(t)
                        msgb[e, pl.ds(k * L, L)] = ex * a
                        exrow[u] = exrow[u] + ex * hot_vecs[k]
                for u, e in enumerate(ee):
                    exb[e, :] = exrow[u]
                return ecarry

            lax.fori_loop(0, CHUNK // 8, edge_body, 0)

        issue_gathers(0, 0)
        issue_gathers(1, jnp.int32(1))

        def pipe_body(i, carry):
            for b in range(2):
                j = 2 * i + b
                wait_gathers(b)

                @pl.when(i >= 1)
                def _():
                    wait_scatters(b)

                compute_chunk(b)
                fill_db(b, j)
                pltpu.async_copy(msg_b[b], numer_sh.at[db_b[b]], sm_b[b],
                                 add=True)
                pltpu.async_copy(ex_b[b], denom_sh.at[db_b[b]], se_b[b],
                                 add=True)
                jn = jnp.minimum(j + 2, cpt - 1)
                issue_gathers(b, jn)
            return carry

        lax.fori_loop(0, cpt // 2, pipe_body, 0)
        for b in range(2):
            wait_gathers(b)
            wait_scatters(b)

        plsc.subcore_barrier()

        bias_vecs = [bias_buf[k, :] for k in range(HL)]
        epsv = jnp.full((L,), 1e-16, jnp.float32)
        idxk = [jnp.full((L,), k, jnp.int32) for k in range(HL)]
        done = 0
        for sz in pieces:
            r0 = row0 + done
            pltpu.sync_copy(numer_sh.at[pl.ds(r0, sz)], xl0.at[pl.ds(0, sz)])
            pltpu.sync_copy(denom_sh.at[pl.ds(r0, sz)], exb0.at[pl.ds(0, sz)])

            def div_body(r, carry):
                den_row = exb0[r, :]
                for k in range(HL):
                    dk = den_row.at[idxk[k]].get(mode="promise_in_bounds")
                    sl = pl.ds(k * L, L)
                    msg0[r, sl] = xl0[r, sl] / (dk + epsv) + bias_vecs[k]
                return carry

            lax.fori_loop(0, sz, div_body, 0)
            pltpu.sync_copy(msg0.at[pl.ds(0, sz)],
                            out_hbm.at[cid, pl.ds(r0, sz)])
            done += sz

    return edge_kernel


def kernel(x, edge_index, W_l, W_r, att, bias):
    E2 = edge_index.shape[1]
    etot = E2 + N
    loop = jnp.arange(N, dtype=edge_index.dtype)
    src = jnp.concatenate([edge_index[0], loop])
    dst = jnp.concatenate([edge_index[1], loop])

    edges_per_tile = -(-etot // (NS * 2 * CHUNK)) * 2 * CHUNK
    cpt = edges_per_tile // CHUNK
    epad = edges_per_tile * NS
    src = jnp.pad(src, (0, epad - etot), constant_values=N)
    dst = jnp.pad(dst, (0, epad - etot), constant_values=N)
    edges_packed = src | (dst << 14)

    xpad = jnp.pad(x, ((0, NXL - N), (0, 0)))
    wcat = jnp.stack([W_l[:, :DH], W_l[:, DH:], W_r[:, :DH], W_r[:, DH:]])
    tab = _project(xpad, wcat)

    att_g = att.reshape(NC, HL, C)
    bias_g = bias.astype(jnp.float32).reshape(NC, HL, C)
    hot = jnp.eye(HL, L, dtype=jnp.float32)
    offs = jnp.stack([
        jnp.full((2, L), 0 * NXL, jnp.int32) + jnp.array([[0], [2 * NXL]], jnp.int32),
        jnp.full((2, L), 1 * NXL, jnp.int32) + jnp.array([[0], [2 * NXL]], jnp.int32),
    ])

    edge_kernel = _make_edge_kernel(cpt)
    out_halves = edge_kernel(tab, edges_packed, att_g, hot, bias_g, offs)

    return out_halves.transpose(1, 0, 2).reshape(NPAD, D)[:N]
